# baseline (speedup 1.0000x reference)
import functools

import jax
import jax.numpy as jnp
from jax.experimental import pallas as pl
from jax.experimental.pallas import tpu as pltpu
from jax.experimental.pallas import tpu_sc as plsc

_MM_ROWS = 128
_CP_ROWS = 64


@jax.jit
def _sc_compact(xp):
    n, _ = xp.shape
    w = 50
    mesh = plsc.VectorSubcoreMesh(core_axis_name="core", subcore_axis_name="subcore")

    @functools.partial(
        pl.kernel,
        out_type=jax.ShapeDtypeStruct((n * w,), jnp.int32),
        mesh=mesh,
        compiler_params=pltpu.CompilerParams(use_tc_tiling_on_sc=False),
    )
    def compact_kernel(xp_hbm, out_hbm):
        def body(i_vmem, o_vmem):
            @pl.loop(0, _CP_ROWS)
            def _(r):
                base = r * w
                for c in (0, 16, 32, w - 16):
                    o_vmem[pl.ds(base + c, 16)] = i_vmem[r, pl.ds(c, 16)]

        pltpu.emit_pipeline(
            body,
            grid=(n // _CP_ROWS,),
            in_specs=[pl.BlockSpec((_CP_ROWS, 128), lambda i: (i, 0))],
            out_specs=[pl.BlockSpec((_CP_ROWS * w,), lambda i: (i,))],
            core_axis_name=("core", "subcore"),
            dimension_semantics=(pltpu.PARALLEL,),
        )(xp_hbm, out_hbm)

    return compact_kernel(xp)


@functools.partial(jax.jit, static_argnums=(2,))
def _sc_gather(table_p, idx_flat, rank):
    num_idx = idx_flat.shape[0]
    mesh = plsc.VectorSubcoreMesh(core_axis_name="core", subcore_axis_name="subcore")

    @functools.partial(
        pl.kernel,
        out_type=jax.ShapeDtypeStruct((num_idx, rank), table_p.dtype),
        mesh=mesh,
        scratch_types=[pltpu.VMEM((128, 128), table_p.dtype)],
        compiler_params=pltpu.CompilerParams(use_tc_tiling_on_sc=False),
    )
    def gather_kernel(tbl_hbm, idx_hbm, out_hbm, rows_v):
        def body(i_vmem, o_vmem):
            pltpu.sync_copy(tbl_hbm.at[i_vmem], rows_v)

            @pl.loop(0, 128)
            def _(r):
                for c in range(0, rank, 16):
                    o_vmem[r, pl.ds(c, 16)] = rows_v[r, pl.ds(c, 16)]

        pltpu.emit_pipeline(
            body,
            grid=(num_idx // 128,),
            in_specs=[pl.BlockSpec((128,), lambda i: (i,))],
            out_specs=[pl.BlockSpec((128, rank), lambda i: (i, 0))],
            core_axis_name=("core", "subcore"),
            dimension_semantics=(pltpu.PARALLEL,),
        )(idx_hbm, out_hbm)

    return gather_kernel(table_p, idx_flat)


def _mm_body(a_ref, b_ref, o_ref):
    br, w, m = o_ref.shape
    res = jnp.dot(a_ref[...], b_ref[...], preferred_element_type=jnp.float32)
    o_ref[...] = res.reshape(br, w, m)


@functools.partial(jax.jit, static_argnums=(2,))
def _tc_project(a, b, n_rows):
    n, k = a.shape
    m = b.shape[1]
    w = n // n_rows
    return pl.pallas_call(
        _mm_body,
        grid=(n_rows // _MM_ROWS,),
        in_specs=[
            pl.BlockSpec((_MM_ROWS * w, k), lambda i: (i, 0)),
            pl.BlockSpec((k, m), lambda i: (0, 0)),
        ],
        out_specs=pl.BlockSpec((_MM_ROWS, w, m), lambda i: (i, 0, 0)),
        out_shape=jax.ShapeDtypeStruct((n_rows, w, m), jnp.float32),
    )(a, b)


def kernel(x, first_factor, last_factor):
    n, w = x.shape
    rank = first_factor.shape[1]
    xp = jnp.pad(x.astype(jnp.int32), ((0, 0), (0, 128 - w)))
    table_p = jnp.pad(first_factor, ((0, 0), (0, 128 - rank)))
    idx_flat = _sc_compact(xp)
    gathered = _sc_gather(table_p, idx_flat, rank)
    return _tc_project(gathered, last_factor, n)


# speedup vs baseline: 1.1896x; 1.1896x over previous
import functools

import jax
import jax.numpy as jnp
from jax.experimental import pallas as pl
from jax.experimental.pallas import tpu as pltpu
from jax.experimental.pallas import tpu_sc as plsc

_MM_ROWS = 128


@jax.jit
def _sc_gather(table, idx_2d):
    rank = table.shape[1]
    n_steps, w = idx_2d.shape
    mesh = plsc.VectorSubcoreMesh(core_axis_name="core", subcore_axis_name="subcore")

    @functools.partial(
        pl.kernel,
        out_type=jax.ShapeDtypeStruct((n_steps * w, rank), table.dtype),
        mesh=mesh,
        compiler_params=pltpu.CompilerParams(use_tc_tiling_on_sc=False),
    )
    def gather_kernel(tbl_hbm, idx_hbm, out_hbm):
        def body(i_vmem, o_vmem):
            pltpu.sync_copy(tbl_hbm.at[i_vmem.at[0]], o_vmem)

        pltpu.emit_pipeline(
            body,
            grid=(n_steps,),
            in_specs=[pl.BlockSpec((1, w), lambda i: (i, 0))],
            out_specs=[pl.BlockSpec((w, rank), lambda i: (i, 0))],
            core_axis_name=("core", "subcore"),
            dimension_semantics=(pltpu.PARALLEL,),
        )(idx_hbm, out_hbm)

    return gather_kernel(table, idx_2d)


def _mm_body(a_ref, b_ref, o_ref):
    br, w, m = o_ref.shape
    res = jnp.dot(a_ref[...], b_ref[...], preferred_element_type=jnp.float32)
    o_ref[...] = res.reshape(br, w, m)


@functools.partial(jax.jit, static_argnums=(2,))
def _tc_project(a, b, n_rows):
    n, k = a.shape
    m = b.shape[1]
    w = n // n_rows
    return pl.pallas_call(
        _mm_body,
        grid=(n_rows // _MM_ROWS,),
        in_specs=[
            pl.BlockSpec((_MM_ROWS * w, k), lambda i: (i, 0)),
            pl.BlockSpec((k, m), lambda i: (0, 0)),
        ],
        out_specs=pl.BlockSpec((_MM_ROWS, w, m), lambda i: (i, 0, 0)),
        out_shape=jax.ShapeDtypeStruct((n_rows, w, m), jnp.float32),
    )(a, b)


def kernel(x, first_factor, last_factor):
    num_idx = x.size
    idx_2d = x.reshape(-1).astype(jnp.int32).reshape(num_idx // 128, 128)
    gathered = _sc_gather(first_factor, idx_2d)
    return _tc_project(gathered, last_factor, x.shape[0])
